# baseline (speedup 1.0000x reference)
import functools

import jax
import jax.numpy as jnp
from jax import lax
from jax.experimental import pallas as pl
from jax.experimental.pallas import tpu as pltpu
from jax.experimental.pallas import tpu_sc as plsc

N = 10000
E = 320000
F = 128
FH = 64
HP = 16
NP = 10112
ROWS = 1264
NB = NP // ROWS
NEG = -1e30

NCORE = 2
NTILE = 16
CB = 128
E_PAD = 327680
CPT = E_PAD // CB // NTILE
MC = 1
MCE = MC * CB
NMC = CPT // MC
NPAIR = NMC // 2
RPT = NP // NTILE

NPS = 10008
HND = NP // 2
DND = HND + 8
DRPT = 320


def _head_matrix(h):
    col = lax.broadcasted_iota(jnp.int32, (F, HP), 1)
    row = lax.broadcasted_iota(jnp.int32, (F, HP), 0)
    d = F // h
    return (row // d == col).astype(jnp.float32)


def _dense_stage_kernel(h, x_ref, w_ref, alf_ref, arf_ref,
                        ftc_ref, elp_ref, erp_ref, elmax_ref, ermax_ref):
    i = pl.program_id(0)
    x = x_ref[...]
    ft = lax.dot_general(x, w_ref[...], (((1,), (1,)), ((), ())),
                         preferred_element_type=jnp.float32)
    ftc_ref[0] = ft[:, :FH]
    ftc_ref[1] = ft[:, FH:]
    sm = _head_matrix(h)
    el = jnp.dot(ft * alf_ref[...], sm, preferred_element_type=jnp.float32)
    er = jnp.dot(ft * arf_ref[...], sm, preferred_element_type=jnp.float32)
    lane = lax.broadcasted_iota(jnp.int32, (ROWS, HP), 1)
    elp = jnp.where(lane < h, el, NEG)
    erp = jnp.where(lane < h, er, NEG)
    elp_ref[...] = elp
    erp_ref[...] = erp
    bmax_l = jnp.max(elp, axis=0, keepdims=True)
    bmax_r = jnp.max(erp, axis=0, keepdims=True)

    @pl.when(i == 0)
    def _():
        elmax_ref[...] = bmax_l
        ermax_ref[...] = bmax_r

    @pl.when(i > 0)
    def _():
        elmax_ref[...] = jnp.maximum(elmax_ref[...], bmax_l)
        ermax_ref[...] = jnp.maximum(ermax_ref[...], bmax_r)


def _dense_stage(x, w, alf, arf, h):
    f32 = jnp.float32
    return pl.pallas_call(
        functools.partial(_dense_stage_kernel, h),
        grid=(NB,),
        in_specs=[
            pl.BlockSpec((ROWS, F), lambda i: (i, 0)),
            pl.BlockSpec((F, F), lambda i: (0, 0)),
            pl.BlockSpec((1, F), lambda i: (0, 0)),
            pl.BlockSpec((1, F), lambda i: (0, 0)),
        ],
        out_specs=[
            pl.BlockSpec((2, ROWS, FH), lambda i: (0, i, 0)),
            pl.BlockSpec((ROWS, HP), lambda i: (i, 0)),
            pl.BlockSpec((ROWS, HP), lambda i: (i, 0)),
            pl.BlockSpec((1, HP), lambda i: (0, 0)),
            pl.BlockSpec((1, HP), lambda i: (0, 0)),
        ],
        out_shape=[
            jax.ShapeDtypeStruct((2, NP, FH), f32),
            jax.ShapeDtypeStruct((NP, HP), f32),
            jax.ShapeDtypeStruct((NP, HP), f32),
            jax.ShapeDtypeStruct((1, HP), f32),
            jax.ShapeDtypeStruct((1, HP), f32),
        ],
    )(x, w, alf, arf)


def _combine_kernel(h, num_ref, den_ref, out_ref):
    num = jnp.concatenate([num_ref[0], num_ref[1]], axis=1)
    den = den_ref[0]
    den = jnp.where(den == 0.0, 1.0, den)
    d = F // h
    parts = []
    for g in range(h):
        r = 1.0 / den[:, g:g + 1]
        parts.append(num[:, g * d:(g + 1) * d] * r)
    out_ref[...] = jnp.concatenate(parts, axis=1)


def _combine(num2, den2, h):
    return pl.pallas_call(
        functools.partial(_combine_kernel, h),
        grid=(NB,),
        in_specs=[
            pl.BlockSpec((2, ROWS, FH), lambda i: (0, i, 0)),
            pl.BlockSpec((1, ROWS, HP), lambda i: (i // 4, i % 4, 0)),
        ],
        out_specs=pl.BlockSpec((ROWS, F), lambda i: (i, 0)),
        out_shape=jax.ShapeDtypeStruct((NP, F), jnp.float32),
    )(num2, den2)


def _edge_sc_body(chg, ghg,
                  ftc_hbm, elp_hbm, erp_hbm, elmax_hbm, ermax_hbm,
                  src2_hbm, dst2_hbm, z64_hbm, z16_hbm,
                  num_hbm, den_hbm,
                  sidx_v, didx_v, m_v,
                  ftb0, ftb1, elb0, elb1, erb0, erb1, dx0, dx1,
                  gsem0, gsem1, ssem0, ssem1,
                  num_sp, den_sp):
    c = lax.axis_index("c")
    s = lax.axis_index("s")
    row0 = s * CPT
    dbase = c * HND
    doff = jnp.minimum(s * DRPT, DND - DRPT)
    noff = jnp.minimum(s * RPT, NPS - RPT)

    pltpu.sync_copy(z64_hbm.at[pl.ds(0, RPT)],
                    num_sp.at[pl.ds(noff, RPT)])
    pltpu.sync_copy(z16_hbm.at[pl.ds(0, DRPT)],
                    den_sp.at[pl.ds(doff, DRPT)])

    @pl.when(s == 0)
    def _():
        pltpu.sync_copy(z64_hbm.at[pl.ds(0, NP - NPS)],
                        num_hbm.at[c, pl.ds(NPS, NP - NPS)])

    pltpu.sync_copy(src2_hbm.at[pl.ds(row0, CPT)], sidx_v)
    pltpu.sync_copy(dst2_hbm.at[pl.ds(row0, CPT)], didx_v)
    pltpu.sync_copy(elmax_hbm, m_v.at[pl.ds(0, 1)])
    pltpu.sync_copy(ermax_hbm, m_v.at[pl.ds(1, 1)])
    msum = m_v[0, :] + m_v[1, :]
    m_v[0, :] = jnp.maximum(msum, 0.2 * msum)

    plsc.subcore_barrier()

    ft_hbm = ftc_hbm.at[c]
    bufs = ((ftb0, elb0, erb0, dx0, gsem0, ssem0),
            (ftb1, elb1, erb1, dx1, gsem1, ssem1))

    def issue_g(m, b):
        ftb, elb, erb, _, gsem, _ = bufs[b]
        for k in range(MC):
            j = m * MC + k
            sl = pl.ds(k * CB, CB)
            pltpu.async_copy(ft_hbm.at[sidx_v.at[j]], ftb.at[sl], gsem)
            pltpu.async_copy(elp_hbm.at[sidx_v.at[j]], elb.at[sl], gsem)
            pltpu.async_copy(erp_hbm.at[didx_v.at[j]], erb.at[sl], gsem)

    def wait_g(b):
        ftb, elb, erb, _, gsem, _ = bufs[b]
        for k in range(MC):
            sl = pl.ds(k * CB, CB)
            pltpu.make_async_copy(ft_hbm.at[pl.ds(0, CB)], ftb.at[sl],
                                  gsem).wait()
            pltpu.make_async_copy(elp_hbm.at[pl.ds(0, CB)], elb.at[sl],
                                  gsem).wait()
            pltpu.make_async_copy(erp_hbm.at[pl.ds(0, CB)], erb.at[sl],
                                  gsem).wait()

    def issue_s(m, b):
        ftb, elb, _, dxb, _, ssem = bufs[b]
        for k in range(MC):
            j = m * MC + k
            sl = pl.ds(k * CB, CB)
            pltpu.async_copy(ftb.at[sl], num_sp.at[didx_v.at[j]], ssem,
                             add=True)
            pltpu.async_copy(elb.at[sl], den_sp.at[dxb.at[k]], ssem,
                             add=True)

    def wait_s(b):
        ftb, elb, _, _, _, ssem = bufs[b]
        for k in range(MC):
            sl = pl.ds(k * CB, CB)
            pltpu.make_async_copy(ftb.at[sl], num_sp.at[pl.ds(0, CB)],
                                  ssem).wait()
            pltpu.make_async_copy(elb.at[sl], den_sp.at[pl.ds(0, CB)],
                                  ssem).wait()

    def compute(m, b):
        ftb, elb, erb, dxb, _, _ = bufs[b]
        mvec = m_v[0, :]

        def edge_loop(cc):
            @pl.loop(0, MCE, unroll=8)
            def _(e):
                x = elb[e, :] + erb[e, :]
                x = jnp.maximum(x, 0.2 * x)
                ee = jnp.exp(x - mvec)
                elb[e, :] = ee
                for g in range(F // HP // 2):
                    hg = cc * chg + g * ghg
                    coef = ee[hg]
                    sl = pl.ds(g * 16, 16)
                    ftb[e, sl] = ftb[e, sl] * coef

        @pl.when(c == 0)
        def _():
            edge_loop(0)

        @pl.when(c == 1)
        def _():
            edge_loop(1)

        for k in range(MC):
            j = m * MC + k
            for r in range(CB // HP):
                sl = pl.ds(r * HP, HP)
                d = didx_v[j, sl] - dbase
                ok = (d >= 0) & (d < HND)
                dxb[k, sl] = jnp.where(ok, d, HND)

    issue_g(0, 0)

    @pl.loop(0, NPAIR)
    def _(p):
        m0 = 2 * p
        wait_g(0)

        @pl.when(p > 0)
        def _():
            wait_s(1)

        issue_g(m0 + 1, 1)
        compute(m0, 0)
        issue_s(m0, 0)
        wait_g(1)
        wait_s(0)

        @pl.when(p < NPAIR - 1)
        def _():
            issue_g(m0 + 2, 0)

        compute(m0 + 1, 1)
        issue_s(m0 + 1, 1)

    wait_s(1)
    plsc.subcore_barrier()
    pltpu.sync_copy(num_sp.at[pl.ds(noff, RPT)],
                    num_hbm.at[c, pl.ds(noff, RPT)])
    pltpu.sync_copy(den_sp.at[pl.ds(doff, DRPT)],
                    den_hbm.at[c, pl.ds(doff, DRPT)])


def _edge_stage(ftc, elp, erp, elmax, ermax, src2, dst2, h):
    f32 = jnp.float32
    chg, ghg = (4, 1) if h == 8 else (0, 0)
    z64 = jnp.zeros((RPT, FH), f32)
    z16 = jnp.zeros((DRPT, HP), f32)
    mesh = plsc.VectorSubcoreMesh(core_axis_name="c", subcore_axis_name="s")
    fn = pl.kernel(
        functools.partial(_edge_sc_body, chg, ghg),
        out_type=[jax.ShapeDtypeStruct((NCORE, NP, FH), f32),
                  jax.ShapeDtypeStruct((NCORE, DND, HP), f32)],
        mesh=mesh,
        compiler_params=pltpu.CompilerParams(use_tc_tiling_on_sc=False),
        scratch_types=[
            pltpu.VMEM((CPT, CB), jnp.int32),
            pltpu.VMEM((CPT, CB), jnp.int32),
            pltpu.VMEM((2, HP), f32),
            pltpu.VMEM((MCE, FH), f32),
            pltpu.VMEM((MCE, FH), f32),
            pltpu.VMEM((MCE, HP), f32),
            pltpu.VMEM((MCE, HP), f32),
            pltpu.VMEM((MCE, HP), f32),
            pltpu.VMEM((MCE, HP), f32),
            pltpu.VMEM((MC, CB), jnp.int32),
            pltpu.VMEM((MC, CB), jnp.int32),
            pltpu.SemaphoreType.DMA,
            pltpu.SemaphoreType.DMA,
            pltpu.SemaphoreType.DMA,
            pltpu.SemaphoreType.DMA,
            pltpu.VMEM_SHARED((NPS, FH), f32),
            pltpu.VMEM_SHARED((DND, HP), f32),
        ],
    )
    return fn(ftc, elp, erp, elmax, ermax, src2, dst2, z64, z16)


def kernel(feats, g, W0, al0, ar0, W1, al1, ar1):
    f32 = jnp.float32
    src2 = jnp.concatenate(
        [g[0], jnp.full((E_PAD - E,), NPS - 1, jnp.int32)]).reshape(-1, CB)
    dst2 = jnp.concatenate(
        [g[1], jnp.full((E_PAD - E,), NPS - 1, jnp.int32)]).reshape(-1, CB)
    feats_p = jnp.zeros((NP, F), f32).at[:N].set(feats)
    alf0 = al0.reshape(1, F)
    arf0 = ar0.reshape(1, F)
    alf1 = al1.reshape(1, F)
    arf1 = ar1.reshape(1, F)

    ftc0, elp0, erp0, elm0, erm0 = _dense_stage(feats_p, W0, alf0, arf0, 8)
    num0, den0 = _edge_stage(ftc0, elp0, erp0, elm0, erm0, src2, dst2, 8)
    h1p = _combine(num0, den0, 8)

    ftc1, elp1, erp1, elm1, erm1 = _dense_stage(h1p, W1, alf1, arf1, 1)
    num1, den1 = _edge_stage(ftc1, elp1, erp1, elm1, erm1, src2, dst2, 1)
    hfp = _combine(num1, den1, 1)
    return (h1p[:N], hfp[:N])


# speedup vs baseline: 1.2934x; 1.2934x over previous
import functools

import jax
import jax.numpy as jnp
from jax import lax
from jax.experimental import pallas as pl
from jax.experimental.pallas import tpu as pltpu
from jax.experimental.pallas import tpu_sc as plsc

N = 10000
E = 320000
F = 128
FH = 64
HP = 16
NP = 10112
ROWS = 1264
NB = NP // ROWS
NEG = -1e30

NCORE = 2
NTILE = 16
CB = 128
E_PAD = 327680
CPT = E_PAD // CB // NTILE
MC = 1
MCE = MC * CB
NMC = CPT // MC
NPAIR = NMC // 2
RPT = NP // NTILE

NPS = 10008
HND = NP // 2
DND = HND + 8
DRPT = 320


def _head_matrix(h):
    col = lax.broadcasted_iota(jnp.int32, (F, HP), 1)
    row = lax.broadcasted_iota(jnp.int32, (F, HP), 0)
    d = F // h
    return (row // d == col).astype(jnp.float32)


def _dense_stage_kernel(h, x_ref, w_ref, alf_ref, arf_ref,
                        ftc_ref, elp_ref, erp_ref, elmax_ref, ermax_ref):
    i = pl.program_id(0)
    x = x_ref[...]
    ft = lax.dot_general(x, w_ref[...], (((1,), (1,)), ((), ())),
                         preferred_element_type=jnp.float32)
    ftc_ref[0] = ft[:, :FH]
    ftc_ref[1] = ft[:, FH:]
    sm = _head_matrix(h)
    el = jnp.dot(ft * alf_ref[...], sm, preferred_element_type=jnp.float32)
    er = jnp.dot(ft * arf_ref[...], sm, preferred_element_type=jnp.float32)
    lane = lax.broadcasted_iota(jnp.int32, (ROWS, HP), 1)
    elp = jnp.where(lane < h, el, NEG)
    erp = jnp.where(lane < h, er, NEG)
    elp_ref[...] = elp
    erp_ref[...] = erp
    bmax_l = jnp.max(elp, axis=0, keepdims=True)
    bmax_r = jnp.max(erp, axis=0, keepdims=True)

    @pl.when(i == 0)
    def _():
        elmax_ref[...] = bmax_l
        ermax_ref[...] = bmax_r

    @pl.when(i > 0)
    def _():
        elmax_ref[...] = jnp.maximum(elmax_ref[...], bmax_l)
        ermax_ref[...] = jnp.maximum(ermax_ref[...], bmax_r)


def _dense_stage(x, w, alf, arf, h):
    f32 = jnp.float32
    return pl.pallas_call(
        functools.partial(_dense_stage_kernel, h),
        grid=(NB,),
        in_specs=[
            pl.BlockSpec((ROWS, F), lambda i: (i, 0)),
            pl.BlockSpec((F, F), lambda i: (0, 0)),
            pl.BlockSpec((1, F), lambda i: (0, 0)),
            pl.BlockSpec((1, F), lambda i: (0, 0)),
        ],
        out_specs=[
            pl.BlockSpec((2, ROWS, FH), lambda i: (0, i, 0)),
            pl.BlockSpec((ROWS, HP), lambda i: (i, 0)),
            pl.BlockSpec((ROWS, HP), lambda i: (i, 0)),
            pl.BlockSpec((1, HP), lambda i: (0, 0)),
            pl.BlockSpec((1, HP), lambda i: (0, 0)),
        ],
        out_shape=[
            jax.ShapeDtypeStruct((2, NP, FH), f32),
            jax.ShapeDtypeStruct((NP, HP), f32),
            jax.ShapeDtypeStruct((NP, HP), f32),
            jax.ShapeDtypeStruct((1, HP), f32),
            jax.ShapeDtypeStruct((1, HP), f32),
        ],
    )(x, w, alf, arf)


def _combine_kernel(h, num_ref, den_ref, out_ref):
    num = jnp.concatenate([num_ref[0], num_ref[1]], axis=1)
    den = den_ref[0]
    den = jnp.where(den == 0.0, 1.0, den)
    d = F // h
    parts = []
    for g in range(h):
        r = 1.0 / den[:, g:g + 1]
        parts.append(num[:, g * d:(g + 1) * d] * r)
    out_ref[...] = jnp.concatenate(parts, axis=1)


def _combine(num2, den2, h):
    return pl.pallas_call(
        functools.partial(_combine_kernel, h),
        grid=(NB,),
        in_specs=[
            pl.BlockSpec((2, ROWS, FH), lambda i: (0, i, 0)),
            pl.BlockSpec((1, ROWS, HP), lambda i: (i // 4, i % 4, 0)),
        ],
        out_specs=pl.BlockSpec((ROWS, F), lambda i: (i, 0)),
        out_shape=jax.ShapeDtypeStruct((NP, F), jnp.float32),
    )(num2, den2)


def _edge_sc_body(chg, ghg,
                  ftc_hbm, elp_hbm, erp_hbm, elmax_hbm, ermax_hbm,
                  src2_hbm, dst2_hbm, z64_hbm, z16_hbm,
                  num_hbm, den_hbm,
                  sidx_v, didx_v, m_v,
                  ftb0, ftb1, elb0, elb1, erb0, erb1, dx0, dx1,
                  gsem0, gsem1, ssem0, ssem1,
                  num_sp, den_sp):
    c = lax.axis_index("c")
    s = lax.axis_index("s")
    row0 = s * CPT
    dbase = c * HND
    doff = jnp.minimum(s * DRPT, DND - DRPT)
    noff = jnp.minimum(s * RPT, NPS - RPT)

    pltpu.sync_copy(z64_hbm.at[pl.ds(0, RPT)],
                    num_sp.at[pl.ds(noff, RPT)])
    pltpu.sync_copy(z16_hbm.at[pl.ds(0, DRPT)],
                    den_sp.at[pl.ds(doff, DRPT)])

    @pl.when(s == 0)
    def _():
        pltpu.sync_copy(z64_hbm.at[pl.ds(0, NP - NPS)],
                        num_hbm.at[c, pl.ds(NPS, NP - NPS)])

    pltpu.sync_copy(src2_hbm.at[pl.ds(row0, CPT)], sidx_v)
    pltpu.sync_copy(dst2_hbm.at[pl.ds(row0, CPT)], didx_v)
    pltpu.sync_copy(elmax_hbm, m_v.at[pl.ds(0, 1)])
    pltpu.sync_copy(ermax_hbm, m_v.at[pl.ds(1, 1)])
    msum = m_v[0, :] + m_v[1, :]
    m_v[0, :] = jnp.maximum(msum, 0.2 * msum)

    plsc.subcore_barrier()

    ft_hbm = ftc_hbm.at[c]
    bufs = ((ftb0, elb0, erb0, dx0, gsem0, ssem0),
            (ftb1, elb1, erb1, dx1, gsem1, ssem1))

    def issue_g(m, b):
        ftb, elb, erb, _, gsem, _ = bufs[b]
        for k in range(MC):
            j = m * MC + k
            sl = pl.ds(k * CB, CB)
            pltpu.async_copy(ft_hbm.at[sidx_v.at[j]], ftb.at[sl], gsem)
            pltpu.async_copy(elp_hbm.at[sidx_v.at[j]], elb.at[sl], gsem)
            pltpu.async_copy(erp_hbm.at[didx_v.at[j]], erb.at[sl], gsem)

    def wait_g(b):
        ftb, elb, erb, _, gsem, _ = bufs[b]
        for k in range(MC):
            sl = pl.ds(k * CB, CB)
            pltpu.make_async_copy(ft_hbm.at[pl.ds(0, CB)], ftb.at[sl],
                                  gsem).wait()
            pltpu.make_async_copy(elp_hbm.at[pl.ds(0, CB)], elb.at[sl],
                                  gsem).wait()
            pltpu.make_async_copy(erp_hbm.at[pl.ds(0, CB)], erb.at[sl],
                                  gsem).wait()

    def issue_s(m, b):
        ftb, elb, _, dxb, _, ssem = bufs[b]
        for k in range(MC):
            j = m * MC + k
            sl = pl.ds(k * CB, CB)
            pltpu.async_copy(ftb.at[sl], num_sp.at[didx_v.at[j]], ssem,
                             add=True)
            pltpu.async_copy(elb.at[sl], den_sp.at[dxb.at[k]], ssem,
                             add=True)

    def wait_s(b):
        ftb, elb, _, _, _, ssem = bufs[b]
        for k in range(MC):
            sl = pl.ds(k * CB, CB)
            pltpu.make_async_copy(ftb.at[sl], num_sp.at[pl.ds(0, CB)],
                                  ssem).wait()
            pltpu.make_async_copy(elb.at[sl], den_sp.at[pl.ds(0, CB)],
                                  ssem).wait()

    def compute(m, b):
        ftb, elb, erb, dxb, _, _ = bufs[b]
        mvec = m_v[0, :]

        def edge_loop(cc):
            @plsc.parallel_loop(0, MCE, unroll=8)
            def _(e):
                x = elb[e, :] + erb[e, :]
                x = jnp.maximum(x, 0.2 * x)
                ee = jnp.exp(x - mvec)
                elb[e, :] = ee
                for g in range(F // HP // 2):
                    hg = cc * chg + g * ghg
                    coef = ee[hg]
                    sl = pl.ds(g * 16, 16)
                    ftb[e, sl] = ftb[e, sl] * coef

        @pl.when(c == 0)
        def _():
            edge_loop(0)

        @pl.when(c == 1)
        def _():
            edge_loop(1)

        for k in range(MC):
            j = m * MC + k
            for r in range(CB // HP):
                sl = pl.ds(r * HP, HP)
                d = didx_v[j, sl] - dbase
                ok = (d >= 0) & (d < HND)
                dxb[k, sl] = jnp.where(ok, d, HND)

    issue_g(0, 0)

    @pl.loop(0, NPAIR)
    def _(p):
        m0 = 2 * p
        wait_g(0)

        @pl.when(p > 0)
        def _():
            wait_s(1)

        issue_g(m0 + 1, 1)
        compute(m0, 0)
        issue_s(m0, 0)
        wait_g(1)
        wait_s(0)

        @pl.when(p < NPAIR - 1)
        def _():
            issue_g(m0 + 2, 0)

        compute(m0 + 1, 1)
        issue_s(m0 + 1, 1)

    wait_s(1)
    plsc.subcore_barrier()
    pltpu.sync_copy(num_sp.at[pl.ds(noff, RPT)],
                    num_hbm.at[c, pl.ds(noff, RPT)])
    pltpu.sync_copy(den_sp.at[pl.ds(doff, DRPT)],
                    den_hbm.at[c, pl.ds(doff, DRPT)])


def _edge_stage(ftc, elp, erp, elmax, ermax, src2, dst2, h):
    f32 = jnp.float32
    chg, ghg = (4, 1) if h == 8 else (0, 0)
    z64 = jnp.zeros((RPT, FH), f32)
    z16 = jnp.zeros((DRPT, HP), f32)
    mesh = plsc.VectorSubcoreMesh(core_axis_name="c", subcore_axis_name="s")
    fn = pl.kernel(
        functools.partial(_edge_sc_body, chg, ghg),
        out_type=[jax.ShapeDtypeStruct((NCORE, NP, FH), f32),
                  jax.ShapeDtypeStruct((NCORE, DND, HP), f32)],
        mesh=mesh,
        compiler_params=pltpu.CompilerParams(use_tc_tiling_on_sc=False),
        scratch_types=[
            pltpu.VMEM((CPT, CB), jnp.int32),
            pltpu.VMEM((CPT, CB), jnp.int32),
            pltpu.VMEM((2, HP), f32),
            pltpu.VMEM((MCE, FH), f32),
            pltpu.VMEM((MCE, FH), f32),
            pltpu.VMEM((MCE, HP), f32),
            pltpu.VMEM((MCE, HP), f32),
            pltpu.VMEM((MCE, HP), f32),
            pltpu.VMEM((MCE, HP), f32),
            pltpu.VMEM((MC, CB), jnp.int32),
            pltpu.VMEM((MC, CB), jnp.int32),
            pltpu.SemaphoreType.DMA,
            pltpu.SemaphoreType.DMA,
            pltpu.SemaphoreType.DMA,
            pltpu.SemaphoreType.DMA,
            pltpu.VMEM_SHARED((NPS, FH), f32),
            pltpu.VMEM_SHARED((DND, HP), f32),
        ],
    )
    return fn(ftc, elp, erp, elmax, ermax, src2, dst2, z64, z16)


def kernel(feats, g, W0, al0, ar0, W1, al1, ar1):
    f32 = jnp.float32
    src2 = jnp.concatenate(
        [g[0], jnp.full((E_PAD - E,), NPS - 1, jnp.int32)]).reshape(-1, CB)
    dst2 = jnp.concatenate(
        [g[1], jnp.full((E_PAD - E,), NPS - 1, jnp.int32)]).reshape(-1, CB)
    feats_p = jnp.zeros((NP, F), f32).at[:N].set(feats)
    alf0 = al0.reshape(1, F)
    arf0 = ar0.reshape(1, F)
    alf1 = al1.reshape(1, F)
    arf1 = ar1.reshape(1, F)

    ftc0, elp0, erp0, elm0, erm0 = _dense_stage(feats_p, W0, alf0, arf0, 8)
    num0, den0 = _edge_stage(ftc0, elp0, erp0, elm0, erm0, src2, dst2, 8)
    h1p = _combine(num0, den0, 8)

    ftc1, elp1, erp1, elm1, erm1 = _dense_stage(h1p, W1, alf1, arf1, 1)
    num1, den1 = _edge_stage(ftc1, elp1, erp1, elm1, erm1, src2, dst2, 1)
    hfp = _combine(num1, den1, 1)
    return (h1p[:N], hfp[:N])
